# baseline (speedup 1.0000x reference)
import jax
import jax.numpy as jnp
from jax.experimental import pallas as pl
from jax.experimental.pallas import tpu as pltpu

H, W = 512, 512
CB = 8


def _snake_block(x_ref, o_ref):
    y = x_ref[0].reshape(CB * H, W)
    n = y.shape[0]
    ridx = 127 - jax.lax.broadcasted_iota(jnp.int32, (n, 128), 1)
    chunks = [
        jnp.take_along_axis(y[:, W - 128 * (j + 1):W - 128 * j], ridx, axis=1)
        for j in range(4)
    ]
    rev = jnp.concatenate(chunks, axis=1)
    r = jax.lax.broadcasted_iota(jnp.int32, y.shape, 0)
    sel = jnp.where((r % 2) == 0, y, rev)
    o_ref[...] = sel.reshape(CB, H * W)


def kernel(x, idx):
    B, C, Hh, Ww = x.shape
    nblk = B * C // CB
    cblk = C // CB
    out = pl.pallas_call(
        _snake_block,
        out_shape=jax.ShapeDtypeStruct((B * C, Hh * Ww), x.dtype),
        grid=(nblk,),
        in_specs=[pl.BlockSpec((1, CB, Hh, Ww),
                               lambda g: (g // cblk, g % cblk, 0, 0))],
        out_specs=pl.BlockSpec((CB, Hh * Ww), lambda g: (g, 0)),
        compiler_params=pltpu.CompilerParams(
            dimension_semantics=("arbitrary",),
        ),
    )(x)
    return out.reshape(B, C, Hh * Ww)
